# baseline (speedup 1.0000x reference)
import jax
import jax.numpy as jnp
from jax import lax
from jax.experimental import pallas as pl
from jax.experimental.pallas import tpu as pltpu
from jax.experimental.pallas import tpu_sc as plsc

N0, N1, N2 = 10000, 30000, 20000
C = 128
NC, NS = 2, 16
NW = NC * NS
W = 128
NSLOT = 4
NG = 2
PADM = NSLOT * W * NW
R = 10000
DUMP = 8
ACC_ROWS = R + DUMP
FZ = 72
FF = 80


def _mm_body(x_ref, *refs):
    k = len(refs) // 2
    x = x_ref[...]
    for w_ref, o_ref in zip(refs[:k], refs[k:]):
        o_ref[...] = jnp.dot(x, w_ref[...], preferred_element_type=jnp.float32)


def _matmuls(x, ws, block=1000):
    n = x.shape[0]
    k = len(ws)
    return pl.pallas_call(
        _mm_body,
        grid=(n // block,),
        in_specs=[pl.BlockSpec((block, C), lambda i: (i, 0))]
        + [pl.BlockSpec((C, C), lambda i: (0, 0))] * k,
        out_specs=[pl.BlockSpec((block, C), lambda i: (i, 0))] * k,
        out_shape=[jax.ShapeDtypeStruct((n, C), jnp.float32)] * k,
    )(x, *ws)


def _merge_body(p_ref, o_ref):
    o_ref[...] = jax.nn.sigmoid(p_ref[0] + p_ref[1])


def _merge_sigmoid(p, block=1000):
    n = p.shape[1]
    return pl.pallas_call(
        _merge_body,
        grid=(n // block,),
        in_specs=[pl.BlockSpec((2, block, C), lambda i: (0, i, 0))],
        out_specs=pl.BlockSpec((block, C), lambda i: (i, 0)),
        out_shape=jax.ShapeDtypeStruct((n, C), jnp.float32),
    )(p)


def _pad_idx(dst, src, n_dst, n_src):
    nnz = dst.shape[0]
    m = (-nnz) % PADM
    dst = jnp.concatenate([dst, jnp.full((m,), n_dst, jnp.int32)])
    src = jnp.concatenate(
        [src, (jnp.arange(m, dtype=jnp.int32) * 997) % n_src])
    return dst, src


_GROUPS = (
    (0, 1, [(0, 0), (1, 1)]),
    (1, 3, [(2, 2), (3, 3), (4, 4)]),
    (2, 2, [(5, 5), (6, 6)]),
)


def _sc_body(*args):
    hs = args[0:7]
    idxs = args[7:21]
    outs = args[21:24]
    p = 24
    ds_ = args[p:p + NSLOT]
    ss_ = args[p + NSLOT:p + 2 * NSLOT]
    gs_ = args[p + 2 * NSLOT:p + 2 * NSLOT + NG]
    acc = args[p + 2 * NSLOT + NG]
    q = p + 2 * NSLOT + NG + 1
    isems = args[q:q + NSLOT]
    gsems = args[q + NSLOT:q + NSLOT + NG]
    ssems = args[q + NSLOT + NG:q + NSLOT + 2 * NG]
    fsem = args[q + NSLOT + 2 * NG]

    cid = lax.axis_index("c")
    sid = lax.axis_index("s")
    wid = cid * NS + sid
    dump = jnp.full((16,), R, jnp.int32) + (sid % DUMP)
    zv = jnp.zeros((16,), jnp.float32)
    zb = gs_[0]

    def run_pass(h, dst_hbm, src_hbm, tbase):
        nwt = dst_hbm.shape[0] // (W * NW)
        w0 = wid * nwt

        def idx_start(i, sl):
            off = (w0 + i) * W
            pltpu.async_copy(dst_hbm.at[pl.ds(off, W)], ds_[sl], isems[sl])
            pltpu.async_copy(src_hbm.at[pl.ds(off, W)], ss_[sl], isems[sl])

        def idx_wait_tr(i, sl):
            off = (w0 + i) * W
            pltpu.make_async_copy(
                dst_hbm.at[pl.ds(off, W)], ds_[sl], isems[sl]).wait()
            pltpu.make_async_copy(
                src_hbm.at[pl.ds(off, W)], ss_[sl], isems[sl]).wait()
            d = ds_[sl]

            def tbody(j, _):
                dd = d[pl.ds(j * 16, 16)]
                l = dd - tbase
                oor = (l < 0) | (l >= R)
                d[pl.ds(j * 16, 16)] = jnp.where(oor, dump, l)
                return 0

            lax.fori_loop(0, W // 16, tbody, 0)

        def g_start(sl, b):
            pltpu.async_copy(h.at[ss_[sl]], gs_[b], gsems[b])

        def g_wait(sl, b):
            pltpu.make_async_copy(h.at[ss_[sl]], gs_[b], gsems[b]).wait()

        def s_start(sl, b):
            pltpu.async_copy(gs_[b], acc.at[ds_[sl]], ssems[b], add=True)

        def s_wait(sl, b):
            pltpu.make_async_copy(gs_[b], acc.at[ds_[sl]], ssems[b]).wait()

        for sl in range(NSLOT):
            idx_start(sl, sl)
        idx_wait_tr(0, 0)
        g_start(0, 0)
        idx_wait_tr(1, 1)
        g_start(1, 1)

        def body(it, _):
            v = it * NSLOT
            idx_wait_tr(v + 2, 2)
            idx_wait_tr(v + 3, 3)
            g_wait(0, 0)
            s_start(0, 0)
            g_wait(1, 1)
            s_start(1, 1)
            s_wait(0, 0)
            g_start(2, 0)
            idx_start(v + 4, 0)
            s_wait(1, 1)
            g_start(3, 1)
            idx_start(v + 5, 1)
            idx_wait_tr(v + 4, 0)
            idx_wait_tr(v + 5, 1)
            g_wait(2, 0)
            s_start(2, 0)
            g_wait(3, 1)
            s_start(3, 1)
            s_wait(2, 0)
            g_start(0, 0)
            idx_start(v + 6, 2)
            s_wait(3, 1)
            g_start(1, 1)
            idx_start(v + 7, 3)
            return 0

        lax.fori_loop(0, nwt // NSLOT - 1, body, 0)
        idx_wait_tr(nwt - 2, 2)
        idx_wait_tr(nwt - 1, 3)
        g_wait(0, 0)
        s_start(0, 0)
        g_wait(1, 1)
        s_start(1, 1)
        s_wait(0, 0)
        g_start(2, 0)
        s_wait(1, 1)
        g_start(3, 1)
        g_wait(2, 0)
        s_start(2, 0)
        g_wait(3, 1)
        s_start(3, 1)
        s_wait(2, 0)
        s_wait(3, 1)

    for out_i, n_ranges, pass_list in _GROUPS:
        out = outs[out_i]

        def task_body(t, _, out=out, pass_list=pass_list):
            tbase = t * R

            def zfill(r, _):
                for j in range(C // 16):
                    zb[r, pl.ds(j * 16, 16)] = zv
                return 0

            lax.fori_loop(0, FZ, zfill, 0)
            nz = ACC_ROWS // FZ
            nz_iter = (nz + NS - 1) // NS

            def zero_body(i, _):
                chunk = i * NS + sid

                @pl.when(chunk < nz)
                def _():
                    pltpu.async_copy(
                        zb.at[pl.ds(0, FZ)],
                        acc.at[pl.ds(chunk * FZ, FZ)], fsem)
                return 0

            def zero_drain(i, _):
                chunk = i * NS + sid

                @pl.when(chunk < nz)
                def _():
                    pltpu.make_async_copy(
                        zb.at[pl.ds(0, FZ)],
                        acc.at[pl.ds(chunk * FZ, FZ)], fsem).wait()
                return 0

            lax.fori_loop(0, nz_iter, zero_body, 0)
            lax.fori_loop(0, nz_iter, zero_drain, 0)
            plsc.subcore_barrier()

            for h_i, idx_i in pass_list:
                run_pass(hs[h_i], idxs[2 * idx_i], idxs[2 * idx_i + 1], tbase)

            plsc.subcore_barrier()

            nf = R // FF
            nf_iter = (nf + NS - 1) // NS

            def flush_body(i, _):
                chunk = i * NS + sid

                @pl.when(chunk < nf)
                def _():
                    pltpu.async_copy(
                        acc.at[pl.ds(chunk * FF, FF)],
                        out.at[cid, pl.ds(tbase + chunk * FF, FF)], fsem)
                return 0

            def flush_drain(i, _):
                chunk = i * NS + sid

                @pl.when(chunk < nf)
                def _():
                    pltpu.make_async_copy(
                        acc.at[pl.ds(chunk * FF, FF)],
                        out.at[cid, pl.ds(tbase + chunk * FF, FF)],
                        fsem).wait()
                return 0

            lax.fori_loop(0, nf_iter, flush_body, 0)
            lax.fori_loop(0, nf_iter, flush_drain, 0)
            plsc.subcore_barrier()
            return 0

        lax.fori_loop(0, n_ranges, task_body, 0)


def _sc_scatter(hs, idx_pairs):
    mesh = plsc.VectorSubcoreMesh(core_axis_name="c", subcore_axis_name="s",
                                  num_cores=NC, num_subcores=NS)
    flat_idx = [a for pair in idx_pairs for a in pair]
    f = pl.kernel(
        _sc_body,
        out_type=[
            jax.ShapeDtypeStruct((NC, N0, C), jnp.float32),
            jax.ShapeDtypeStruct((NC, N1, C), jnp.float32),
            jax.ShapeDtypeStruct((NC, N2, C), jnp.float32),
        ],
        mesh=mesh,
        scratch_types=(
            [pltpu.VMEM((W,), jnp.int32)] * NSLOT
            + [pltpu.VMEM((W,), jnp.int32)] * NSLOT
            + [pltpu.VMEM((W, C), jnp.float32)] * NG
            + [pltpu.VMEM_SHARED((ACC_ROWS, C), jnp.float32)]
            + [pltpu.SemaphoreType.DMA] * (NSLOT + 2 * NG + 1)
        ),
    )
    return f(*hs, *flat_idx)


@jax.jit
def kernel(x0, x1, x2, adj0_idx, adj1_idx, adj2_idx, inc1_idx, inc2_idx,
           W_same_0, W_same_1, W_same_2, W_h2l_0, W_h2l_1, W_l2h_1, W_l2h_2):
    h_s0, h_l2h1 = _matmuls(x0, [W_same_0, W_l2h_1])
    h_s1, h_h2l0, h_l2h2 = _matmuls(x1, [W_same_1, W_h2l_0, W_l2h_2])
    h_s2, h_h2l1 = _matmuls(x2, [W_same_2, W_h2l_1])

    idx_pairs = [
        _pad_idx(adj0_idx[0], adj0_idx[1], N0, N0),
        _pad_idx(inc1_idx[0], inc1_idx[1], N0, N1),
        _pad_idx(adj1_idx[0], adj1_idx[1], N1, N1),
        _pad_idx(inc2_idx[0], inc2_idx[1], N1, N2),
        _pad_idx(inc1_idx[1], inc1_idx[0], N1, N0),
        _pad_idx(adj2_idx[0], adj2_idx[1], N2, N2),
        _pad_idx(inc2_idx[1], inc2_idx[0], N2, N1),
    ]
    hs = (h_s0, h_h2l0, h_s1, h_h2l1, h_l2h1, h_s2, h_l2h2)
    p0, p1, p2 = _sc_scatter(hs, idx_pairs)
    return (_merge_sigmoid(p0), _merge_sigmoid(p1), _merge_sigmoid(p2))


# speedup vs baseline: 1.1653x; 1.1653x over previous
import jax
import jax.numpy as jnp
from jax import lax
from jax.experimental import pallas as pl
from jax.experimental.pallas import tpu as pltpu
from jax.experimental.pallas import tpu_sc as plsc

N0, N1, N2 = 10000, 30000, 20000
C = 128
NC, NS = 2, 16
NW = NC * NS
W = 112
NG = 3
PADM = 2 * NG * W * NW
R = 10000
DUMP = 8
ACC_ROWS = R + DUMP
FZ = 72
FF = 80


def _mm_body(x_ref, *refs):
    k = len(refs) // 2
    x = x_ref[...]
    for w_ref, o_ref in zip(refs[:k], refs[k:]):
        o_ref[...] = jnp.dot(x, w_ref[...], preferred_element_type=jnp.float32)


def _matmuls(x, ws, block=1000):
    n = x.shape[0]
    k = len(ws)
    return pl.pallas_call(
        _mm_body,
        grid=(n // block,),
        in_specs=[pl.BlockSpec((block, C), lambda i: (i, 0))]
        + [pl.BlockSpec((C, C), lambda i: (0, 0))] * k,
        out_specs=[pl.BlockSpec((block, C), lambda i: (i, 0))] * k,
        out_shape=[jax.ShapeDtypeStruct((n, C), jnp.float32)] * k,
    )(x, *ws)


def _merge_body(p_ref, o_ref):
    o_ref[...] = jax.nn.sigmoid(p_ref[0] + p_ref[1])


def _merge_sigmoid(p, block=1000):
    n = p.shape[1]
    return pl.pallas_call(
        _merge_body,
        grid=(n // block,),
        in_specs=[pl.BlockSpec((2, block, C), lambda i: (0, i, 0))],
        out_specs=pl.BlockSpec((block, C), lambda i: (i, 0)),
        out_shape=jax.ShapeDtypeStruct((n, C), jnp.float32),
    )(p)


def _pad_idx(dst, src, n_dst, n_src):
    nnz = dst.shape[0]
    m = (-nnz) % PADM
    dst = jnp.concatenate([dst, jnp.full((m,), n_dst, jnp.int32)])
    src = jnp.concatenate(
        [src, (jnp.arange(m, dtype=jnp.int32) * 997) % n_src])
    return dst, src


_GROUPS = (
    (0, 1, [(0, 0), (1, 1)]),
    (1, 3, [(2, 2), (3, 3), (4, 4)]),
    (2, 2, [(5, 5), (6, 6)]),
)


def _sc_body(*args):
    hs = args[0:7]
    idxs = args[7:21]
    outs = args[21:24]
    p = 24
    dA, sA, dB, sB = args[p:p + 4]
    gs_ = args[p + 4:p + 4 + NG]
    acc = args[p + 4 + NG]
    q = p + 4 + NG + 1
    isemA, isemB = args[q:q + 2]
    gsems = args[q + 2:q + 2 + NG]
    ssems = args[q + 2 + NG:q + 2 + 2 * NG]
    fsem = args[q + 2 + 2 * NG]

    cid = lax.axis_index("c")
    sid = lax.axis_index("s")
    wid = cid * NS + sid
    dump = jnp.full((16,), R, jnp.int32) + (sid % DUMP)
    zv = jnp.zeros((16,), jnp.float32)
    zb = gs_[0]

    def run_pass(h, dst_hbm, src_hbm, tbase):
        nbt = dst_hbm.shape[0] // (NG * W * NW)
        b0 = wid * nbt

        def istart(i, d, s, sem):
            off = (b0 + i) * NG * W
            for b in range(NG):
                pltpu.async_copy(
                    dst_hbm.at[pl.ds(off + b * W, W)], d.at[b], sem)
                pltpu.async_copy(
                    src_hbm.at[pl.ds(off + b * W, W)], s.at[b], sem)

        def iwait_tr(i, d, s, sem):
            off = (b0 + i) * NG * W
            for b in range(NG):
                pltpu.make_async_copy(
                    dst_hbm.at[pl.ds(off + b * W, W)], d.at[b], sem).wait()
                pltpu.make_async_copy(
                    src_hbm.at[pl.ds(off + b * W, W)], s.at[b], sem).wait()

            def tbody(j, _):
                for b in range(NG):
                    dd = d[b, pl.ds(j * 16, 16)]
                    l = dd - tbase
                    oor = (l < 0) | (l >= R)
                    d[b, pl.ds(j * 16, 16)] = jnp.where(oor, dump, l)
                return 0

            lax.fori_loop(0, W // 16, tbody, 0)

        def g_start(s, b):
            pltpu.async_copy(h.at[s.at[b]], gs_[b], gsems[b])

        def g_wait(s, b):
            pltpu.make_async_copy(h.at[s.at[b]], gs_[b], gsems[b]).wait()

        def s_start(d, b):
            pltpu.async_copy(gs_[b], acc.at[d.at[b]], ssems[b], add=True)

        def s_wait(d, b):
            pltpu.make_async_copy(gs_[b], acc.at[d.at[b]], ssems[b]).wait()

        istart(0, dA, sA, isemA)
        iwait_tr(0, dA, sA, isemA)
        for b in range(NG):
            g_start(sA, b)
        istart(1, dB, sB, isemB)

        def body(it, _):
            v = 2 * it
            iwait_tr(v + 1, dB, sB, isemB)
            for b in range(NG):
                g_wait(sA, b)
                s_start(dA, b)
            for b in range(NG):
                s_wait(dA, b)
                g_start(sB, b)
            istart(v + 2, dA, sA, isemA)
            iwait_tr(v + 2, dA, sA, isemA)
            for b in range(NG):
                g_wait(sB, b)
                s_start(dB, b)
            for b in range(NG):
                s_wait(dB, b)
                g_start(sA, b)
            istart(v + 3, dB, sB, isemB)
            return 0

        lax.fori_loop(0, nbt // 2 - 1, body, 0)
        iwait_tr(nbt - 1, dB, sB, isemB)
        for b in range(NG):
            g_wait(sA, b)
            s_start(dA, b)
        for b in range(NG):
            s_wait(dA, b)
            g_start(sB, b)
        for b in range(NG):
            g_wait(sB, b)
            s_start(dB, b)
        for b in range(NG):
            s_wait(dB, b)

    for out_i, n_ranges, pass_list in _GROUPS:
        out = outs[out_i]

        def task_body(t, _, out=out, pass_list=pass_list):
            tbase = t * R

            def zfill(r, _):
                for j in range(C // 16):
                    zb[r, pl.ds(j * 16, 16)] = zv
                return 0

            lax.fori_loop(0, FZ, zfill, 0)
            nz = ACC_ROWS // FZ
            nz_iter = (nz + NS - 1) // NS

            def zero_body(i, _):
                chunk = i * NS + sid

                @pl.when(chunk < nz)
                def _():
                    pltpu.async_copy(
                        zb.at[pl.ds(0, FZ)],
                        acc.at[pl.ds(chunk * FZ, FZ)], fsem)
                return 0

            def zero_drain(i, _):
                chunk = i * NS + sid

                @pl.when(chunk < nz)
                def _():
                    pltpu.make_async_copy(
                        zb.at[pl.ds(0, FZ)],
                        acc.at[pl.ds(chunk * FZ, FZ)], fsem).wait()
                return 0

            lax.fori_loop(0, nz_iter, zero_body, 0)
            lax.fori_loop(0, nz_iter, zero_drain, 0)
            plsc.subcore_barrier()

            for h_i, idx_i in pass_list:
                run_pass(hs[h_i], idxs[2 * idx_i], idxs[2 * idx_i + 1], tbase)

            plsc.subcore_barrier()

            nf = R // FF
            nf_iter = (nf + NS - 1) // NS

            def flush_body(i, _):
                chunk = i * NS + sid

                @pl.when(chunk < nf)
                def _():
                    pltpu.async_copy(
                        acc.at[pl.ds(chunk * FF, FF)],
                        out.at[cid, pl.ds(tbase + chunk * FF, FF)], fsem)
                return 0

            def flush_drain(i, _):
                chunk = i * NS + sid

                @pl.when(chunk < nf)
                def _():
                    pltpu.make_async_copy(
                        acc.at[pl.ds(chunk * FF, FF)],
                        out.at[cid, pl.ds(tbase + chunk * FF, FF)],
                        fsem).wait()
                return 0

            lax.fori_loop(0, nf_iter, flush_body, 0)
            lax.fori_loop(0, nf_iter, flush_drain, 0)
            plsc.subcore_barrier()
            return 0

        lax.fori_loop(0, n_ranges, task_body, 0)


def _sc_scatter(hs, idx_pairs):
    mesh = plsc.VectorSubcoreMesh(core_axis_name="c", subcore_axis_name="s",
                                  num_cores=NC, num_subcores=NS)
    flat_idx = [a for pair in idx_pairs for a in pair]
    f = pl.kernel(
        _sc_body,
        out_type=[
            jax.ShapeDtypeStruct((NC, N0, C), jnp.float32),
            jax.ShapeDtypeStruct((NC, N1, C), jnp.float32),
            jax.ShapeDtypeStruct((NC, N2, C), jnp.float32),
        ],
        mesh=mesh,
        scratch_types=(
            [pltpu.VMEM((NG, W), jnp.int32)] * 4
            + [pltpu.VMEM((W, C), jnp.float32)] * NG
            + [pltpu.VMEM_SHARED((ACC_ROWS, C), jnp.float32)]
            + [pltpu.SemaphoreType.DMA] * (2 + 2 * NG + 1)
        ),
    )
    return f(*hs, *flat_idx)


@jax.jit
def kernel(x0, x1, x2, adj0_idx, adj1_idx, adj2_idx, inc1_idx, inc2_idx,
           W_same_0, W_same_1, W_same_2, W_h2l_0, W_h2l_1, W_l2h_1, W_l2h_2):
    h_s0, h_l2h1 = _matmuls(x0, [W_same_0, W_l2h_1])
    h_s1, h_h2l0, h_l2h2 = _matmuls(x1, [W_same_1, W_h2l_0, W_l2h_2])
    h_s2, h_h2l1 = _matmuls(x2, [W_same_2, W_h2l_1])

    idx_pairs = [
        _pad_idx(adj0_idx[0], adj0_idx[1], N0, N0),
        _pad_idx(inc1_idx[0], inc1_idx[1], N0, N1),
        _pad_idx(adj1_idx[0], adj1_idx[1], N1, N1),
        _pad_idx(inc2_idx[0], inc2_idx[1], N1, N2),
        _pad_idx(inc1_idx[1], inc1_idx[0], N1, N0),
        _pad_idx(adj2_idx[0], adj2_idx[1], N2, N2),
        _pad_idx(inc2_idx[1], inc2_idx[0], N2, N1),
    ]
    hs = (h_s0, h_h2l0, h_s1, h_h2l1, h_l2h1, h_s2, h_l2h2)
    p0, p1, p2 = _sc_scatter(hs, idx_pairs)
    return (_merge_sigmoid(p0), _merge_sigmoid(p1), _merge_sigmoid(p2))
